# Initial kernel scaffold; baseline (speedup 1.0000x reference)
import jax
import jax.numpy as jnp
from jax.experimental import pallas as pl


def kernel(input, target, inds, mask):
    raise NotImplementedError("write your pallas kernel here")


# speedup vs baseline: 1.5714x; 1.5714x over previous
import functools

import jax
import jax.numpy as jnp
from jax import lax
from jax.experimental import pallas as pl
from jax.experimental.pallas import tpu as pltpu
from jax.experimental.pallas import tpu_sc as plsc

_L = 16
_NC, _NS = 2, 16


def _make_sc_kernel(B, C, KHP):
    NW = _NC * _NS
    NG = KHP // _L

    mesh = plsc.VectorSubcoreMesh(core_axis_name="c", subcore_axis_name="s")

    @functools.partial(
        pl.kernel,
        mesh=mesh,
        out_type=(
            jax.ShapeDtypeStruct((NW, _L), jnp.float32),
            jax.ShapeDtypeStruct((NW, _L), jnp.float32),
        ),
        scratch_types=[
            pltpu.VMEM((C * KHP,), jnp.int32),
            pltpu.VMEM((C * KHP,), jnp.float32),
            pltpu.VMEM((C * KHP,), jnp.float32),
            pltpu.VMEM((KHP,), jnp.float32),
            pltpu.VMEM((_L,), jnp.float32),
            pltpu.VMEM((_L,), jnp.float32),
            pltpu.SemaphoreType.DMA,
        ],
    )
    def k(idx_hbm, tgt_hbm, maskp_hbm, inflat_hbm, psum_hbm, msum_hbm,
          idx_v, tgt_v, g_v, mask_v, ps_v, ms_v, sem):
        cid = lax.axis_index("c")
        sid = lax.axis_index("s")
        wid = sid * _NC + cid
        b = sid
        kh = cid

        pltpu.sync_copy(idx_hbm.at[b, kh], idx_v)
        pltpu.sync_copy(tgt_hbm.at[b, kh], tgt_v)
        pltpu.sync_copy(maskp_hbm.at[b, kh], mask_v)
        pltpu.async_copy(inflat_hbm.at[idx_v], g_v, sem).wait()

        def group_body(i, carry):
            acc, msum = carry
            base = i * _L

            def chan_body(c, gsum):
                o = c * KHP + base
                return gsum + jnp.abs(g_v[pl.ds(o, _L)] - tgt_v[pl.ds(o, _L)])

            gsum = lax.fori_loop(0, C, chan_body, jnp.zeros((_L,), jnp.float32))
            m = mask_v[pl.ds(base, _L)]
            return acc + gsum * jnp.abs(m), msum + m

        acc, msum = lax.fori_loop(
            0, NG, group_body,
            (jnp.zeros((_L,), jnp.float32), jnp.zeros((_L,), jnp.float32)))

        ps_v[...] = acc
        ms_v[...] = msum
        pltpu.sync_copy(ps_v, psum_hbm.at[wid])
        pltpu.sync_copy(ms_v, msum_hbm.at[wid])

    return k


def kernel(input, target, inds, mask):
    B, C, H, W = input.shape
    K = target.shape[1]
    HW = H * W

    KH = K // _NC
    KHP = ((KH + _L - 1) // _L) * _L

    inds32 = inds.astype(jnp.int32).reshape(B, _NC, KH)
    inds_p = jnp.pad(inds32, ((0, 0), (0, 0), (0, KHP - KH)))
    mask_p = jnp.pad(mask.reshape(B, _NC, KH), ((0, 0), (0, 0), (0, KHP - KH)))

    plane = (jnp.arange(B, dtype=jnp.int32)[:, None, None, None] * C
             + jnp.arange(C, dtype=jnp.int32)[None, None, :, None]) * HW
    idx_cm = (plane + inds_p[:, :, None, :]).reshape(B, _NC, C * KHP)

    tgt_cm = jnp.pad(
        target.astype(jnp.float32).reshape(B, _NC, KH, C).transpose(0, 1, 3, 2),
        ((0, 0), (0, 0), (0, 0), (0, KHP - KH))).reshape(B, _NC, C * KHP)

    input_flat = input.reshape(-1)

    k = _make_sc_kernel(B, C, KHP)
    psum, msum = k(idx_cm, tgt_cm, mask_p, input_flat)
    return psum.sum() / msum.sum()
